# Initial kernel scaffold; baseline (speedup 1.0000x reference)
import jax
import jax.numpy as jnp
from jax.experimental import pallas as pl


def kernel(unary, deltas, index1, index2):
    raise NotImplementedError("write your pallas kernel here")


# speedup vs baseline: 3.6597x; 3.6597x over previous
import functools

import jax
import jax.numpy as jnp
from jax import lax
from jax.experimental import pallas as pl
from jax.experimental.pallas import tpu as pltpu
from jax.experimental.pallas import tpu_sc as plsc

N = 131072
U = 64
NFLAT = N * U

NCORES = 2
NSUB = 16

ELEMS_PER_TILE = NFLAT // NSUB
CHUNK_ELEMS = 32768
NCHUNKS = ELEMS_PER_TILE // CHUNK_ELEMS

MSET_PER_TILE = NFLAT // NSUB
MSET_BUF = 16384
MSET_STEPS = MSET_PER_TILE // MSET_BUF


def _sc_mask_body(idx1_hbm, idx2_hbm, m1_hbm, m2_hbm, buf, zbuf, ones, sem):
    c = lax.axis_index("c")
    s = lax.axis_index("s")

    def _fill(i, _):
        ones[pl.ds(i * 16, 16)] = jnp.full((16,), 1.0, jnp.float32)
        return _
    lax.fori_loop(0, MSET_BUF // 16, _fill, None)

    def _zfill(i, _):
        zbuf[pl.ds(i * 16, 16)] = jnp.zeros((16,), jnp.float32)
        return _
    lax.fori_loop(0, CHUNK_ELEMS // 16, _zfill, None)

    iota = lax.iota(jnp.int32, 16)
    jvecs = [iota + g * 16 for g in range(4)]

    def _work(idx_hbm, m_hbm):
        base = s * MSET_PER_TILE

        def _mset(k, _):
            pltpu.sync_copy(ones, m_hbm.at[pl.ds(base + k * MSET_BUF, MSET_BUF)])
            return _
        lax.fori_loop(0, MSET_STEPS, _mset, None)

        plsc.subcore_barrier()

        for chunk in range(NCHUNKS):
            e0 = s * ELEMS_PER_TILE + chunk * CHUNK_ELEMS
            pltpu.sync_copy(idx_hbm.at[pl.ds(e0, CHUNK_ELEMS)], buf)

            def _addr(i, _):
                base = i * 64
                for g in range(4):
                    v = buf[pl.ds(base + g * 16, 16)]
                    buf[pl.ds(base + g * 16, 16)] = v * U + jvecs[g]
                return _
            lax.fori_loop(0, CHUNK_ELEMS // 64, _addr, None)

            pltpu.async_copy(zbuf, m_hbm.at[buf], sem).wait()

    @pl.when(c == 0)
    def _():
        _work(idx1_hbm, m1_hbm)

    @pl.when(c == 1)
    def _():
        _work(idx2_hbm, m2_hbm)


@jax.jit
def _sc_masks(idx1_flat, idx2_flat):
    return pl.kernel(
        _sc_mask_body,
        mesh=plsc.VectorSubcoreMesh(core_axis_name="c", subcore_axis_name="s"),
        out_type=[
            jax.ShapeDtypeStruct((NFLAT,), jnp.float32),
            jax.ShapeDtypeStruct((NFLAT,), jnp.float32),
        ],
        scratch_types=[
            pltpu.VMEM((CHUNK_ELEMS,), jnp.int32),
            pltpu.VMEM((CHUNK_ELEMS,), jnp.float32),
            pltpu.VMEM((MSET_BUF,), jnp.float32),
            pltpu.SemaphoreType.DMA,
        ],
    )(idx1_flat, idx2_flat)


def _tc_combine_body(deltas_ref, m1_ref, m2_ref, out1_ref, b_ref):
    d = deltas_ref[...]
    out1_ref[...] = d[:, :U] * m1_ref[...] + d[:, U:2 * U] * m2_ref[...]
    b_ref[...] = d[:, 2 * U:]


@jax.jit
def _tc_combine(deltas, m1, m2):
    rows = 512
    grid = (N // rows,)
    return pl.pallas_call(
        _tc_combine_body,
        grid=grid,
        in_specs=[
            pl.BlockSpec((rows, 3 * U), lambda i: (i, 0)),
            pl.BlockSpec((rows, U), lambda i: (i, 0)),
            pl.BlockSpec((rows, U), lambda i: (i, 0)),
        ],
        out_specs=[
            pl.BlockSpec((rows, U), lambda i: (i, 0)),
            pl.BlockSpec((rows, U), lambda i: (i, 0)),
        ],
        out_shape=[
            jax.ShapeDtypeStruct((N, U), jnp.float32),
            jax.ShapeDtypeStruct((N, U), jnp.float32),
        ],
        compiler_params=pltpu.CompilerParams(
            dimension_semantics=("arbitrary",),
        ),
    )(deltas, m1, m2)


def kernel(unary, deltas, index1, index2):
    m1f, m2f = _sc_masks(index1.reshape(NFLAT), index2.reshape(NFLAT))
    out1, b = _tc_combine(deltas, m1f.reshape(N, U), m2f.reshape(N, U))
    return (out1, b)


# speedup vs baseline: 3.6599x; 1.0001x over previous
import functools

import jax
import jax.numpy as jnp
from jax import lax
from jax.experimental import pallas as pl
from jax.experimental.pallas import tpu as pltpu
from jax.experimental.pallas import tpu_sc as plsc

N = 131072
U = 64
NFLAT = N * U

NCORES = 2
NSUB = 16

ELEMS_PER_TILE = NFLAT // NSUB
CHUNK_ELEMS = 16384
NCHUNKS = ELEMS_PER_TILE // CHUNK_ELEMS
NRING = 4

MSET_PER_TILE = NFLAT // NSUB
MSET_BUF = 16384
MSET_STEPS = MSET_PER_TILE // MSET_BUF


def _sc_mask_body(idx1_hbm, idx2_hbm, m1_hbm, m2_hbm,
                  buf0, buf1, buf2, buf3, zbuf, ones, sems):
    bufs = [buf0, buf1, buf2, buf3]
    c = lax.axis_index("c")
    s = lax.axis_index("s")

    def _fill(i, _):
        ones[pl.ds(i * 16, 16)] = jnp.full((16,), 1.0, jnp.float32)
        return _
    lax.fori_loop(0, MSET_BUF // 16, _fill, None)

    def _zfill(i, _):
        zbuf[pl.ds(i * 16, 16)] = jnp.zeros((16,), jnp.float32)
        return _
    lax.fori_loop(0, CHUNK_ELEMS // 16, _zfill, None)

    iota = lax.iota(jnp.int32, 16)
    jvecs = [iota + g * 16 for g in range(4)]

    def _work(idx_hbm, m_hbm):
        base = s * MSET_PER_TILE

        def _mset(k, _):
            pltpu.sync_copy(ones, m_hbm.at[pl.ds(base + k * MSET_BUF, MSET_BUF)])
            return _
        lax.fori_loop(0, MSET_STEPS, _mset, None)

        plsc.subcore_barrier()

        handles = [None] * NRING
        for chunk in range(NCHUNKS):
            slot = chunk % NRING
            if handles[slot] is not None:
                handles[slot].wait()
            buf = bufs[slot]
            e0 = s * ELEMS_PER_TILE + chunk * CHUNK_ELEMS
            pltpu.sync_copy(idx_hbm.at[pl.ds(e0, CHUNK_ELEMS)], buf)

            def _addr(i, _):
                base = i * 64
                for g in range(4):
                    v = buf[pl.ds(base + g * 16, 16)]
                    buf[pl.ds(base + g * 16, 16)] = v * U + jvecs[g]
                return _
            lax.fori_loop(0, CHUNK_ELEMS // 64, _addr, None)

            handles[slot] = pltpu.async_copy(zbuf, m_hbm.at[buf], sems.at[slot])
        for h in handles:
            h.wait()

    @pl.when(c == 0)
    def _():
        _work(idx1_hbm, m1_hbm)

    @pl.when(c == 1)
    def _():
        _work(idx2_hbm, m2_hbm)


@jax.jit
def _sc_masks(idx1_flat, idx2_flat):
    return pl.kernel(
        _sc_mask_body,
        mesh=plsc.VectorSubcoreMesh(core_axis_name="c", subcore_axis_name="s"),
        out_type=[
            jax.ShapeDtypeStruct((NFLAT,), jnp.float32),
            jax.ShapeDtypeStruct((NFLAT,), jnp.float32),
        ],
        scratch_types=[
            pltpu.VMEM((CHUNK_ELEMS,), jnp.int32),
            pltpu.VMEM((CHUNK_ELEMS,), jnp.int32),
            pltpu.VMEM((CHUNK_ELEMS,), jnp.int32),
            pltpu.VMEM((CHUNK_ELEMS,), jnp.int32),
            pltpu.VMEM((CHUNK_ELEMS,), jnp.float32),
            pltpu.VMEM((MSET_BUF,), jnp.float32),
            pltpu.SemaphoreType.DMA((NRING,)),
        ],
    )(idx1_flat, idx2_flat)


def _tc_combine_body(deltas_ref, m1_ref, m2_ref, out1_ref, b_ref):
    d = deltas_ref[...]
    out1_ref[...] = d[:, :U] * m1_ref[...] + d[:, U:2 * U] * m2_ref[...]
    b_ref[...] = d[:, 2 * U:]


@jax.jit
def _tc_combine(deltas, m1, m2):
    rows = 512
    grid = (N // rows,)
    return pl.pallas_call(
        _tc_combine_body,
        grid=grid,
        in_specs=[
            pl.BlockSpec((rows, 3 * U), lambda i: (i, 0)),
            pl.BlockSpec((rows, U), lambda i: (i, 0)),
            pl.BlockSpec((rows, U), lambda i: (i, 0)),
        ],
        out_specs=[
            pl.BlockSpec((rows, U), lambda i: (i, 0)),
            pl.BlockSpec((rows, U), lambda i: (i, 0)),
        ],
        out_shape=[
            jax.ShapeDtypeStruct((N, U), jnp.float32),
            jax.ShapeDtypeStruct((N, U), jnp.float32),
        ],
        compiler_params=pltpu.CompilerParams(
            dimension_semantics=("arbitrary",),
        ),
    )(deltas, m1, m2)


def kernel(unary, deltas, index1, index2):
    m1f, m2f = _sc_masks(index1.reshape(NFLAT), index2.reshape(NFLAT))
    out1, b = _tc_combine(deltas, m1f.reshape(N, U), m2f.reshape(N, U))
    return (out1, b)


# speedup vs baseline: 46.7496x; 12.7734x over previous
import jax
import jax.numpy as jnp
from jax import lax
from jax.experimental import pallas as pl
from jax.experimental.pallas import tpu as pltpu
from jax.experimental.pallas import tpu_sc as plsc

N = 131072
U = 64
NCORES = 2
NSUB = 16
NWORKERS = NCORES * NSUB
HALF = N // 2
TASKS_PER_WORKER = U * 2 // NWORKERS
ICHUNK = 16384
NICHUNK = N // ICHUNK


def _tc_pre_body(deltas_ref, idx1_ref, idx2_ref,
                 uxt_ref, uyt_ref, b_ref, i1t_ref, i2t_ref):
    d = deltas_ref[...]
    uxt_ref[...] = d[:, :U].T
    uyt_ref[...] = d[:, U:2 * U].T
    b_ref[...] = d[:, 2 * U:]
    i1t_ref[...] = idx1_ref[...].T
    i2t_ref[...] = idx2_ref[...].T


@jax.jit
def _tc_pre(deltas, index1, index2):
    rows = 512
    grid = (N // rows,)
    return pl.pallas_call(
        _tc_pre_body,
        grid=grid,
        in_specs=[
            pl.BlockSpec((rows, 3 * U), lambda i: (i, 0)),
            pl.BlockSpec((rows, U), lambda i: (i, 0)),
            pl.BlockSpec((rows, U), lambda i: (i, 0)),
        ],
        out_specs=[
            pl.BlockSpec((U, rows), lambda i: (0, i)),
            pl.BlockSpec((U, rows), lambda i: (0, i)),
            pl.BlockSpec((rows, U), lambda i: (i, 0)),
            pl.BlockSpec((U, rows), lambda i: (0, i)),
            pl.BlockSpec((U, rows), lambda i: (0, i)),
        ],
        out_shape=[
            jax.ShapeDtypeStruct((U, N), jnp.float32),
            jax.ShapeDtypeStruct((U, N), jnp.float32),
            jax.ShapeDtypeStruct((N, U), jnp.float32),
            jax.ShapeDtypeStruct((U, N), jnp.int32),
            jax.ShapeDtypeStruct((U, N), jnp.int32),
        ],
        compiler_params=pltpu.CompilerParams(
            dimension_semantics=("arbitrary",),
        ),
    )(deltas, index1, index2)


def _sc_body(uxt_hbm, uyt_hbm, i1t_hbm, i2t_hbm, at_hbm, bt_hbm,
             dbuf, ibuf0, ibuf1, sems):
    w = lax.axis_index("s") * NCORES + lax.axis_index("c")
    zeros16 = jnp.zeros((16,), jnp.float32)
    ibufs = [ibuf0, ibuf1]

    def _sub_task(src_hbm, idx_hbm, dst_hbm, j, half):
        lo = half * HALF
        pltpu.sync_copy(src_hbm.at[j, pl.ds(lo, HALF)], dbuf)
        h = pltpu.async_copy(idx_hbm.at[j, pl.ds(0, ICHUNK)], ibufs[0],
                             sems.at[0])
        handles = [h, None]
        for k in range(NICHUNK):
            if k + 1 < NICHUNK:
                handles[(k + 1) % 2] = pltpu.async_copy(
                    idx_hbm.at[j, pl.ds((k + 1) * ICHUNK, ICHUNK)],
                    ibufs[(k + 1) % 2], sems.at[(k + 1) % 2])
            handles[k % 2].wait()
            ibuf = ibufs[k % 2]

            def _scan(i, _):
                r = ibuf[pl.ds(i * 16, 16)]
                local = r - lo
                ok = (local >= 0) & (local < HALF)
                plsc.store_scatter(dbuf, [local], zeros16, mask=ok)
                return _
            lax.fori_loop(0, ICHUNK // 16, _scan, None)
        pltpu.sync_copy(dbuf, dst_hbm.at[j, pl.ds(lo, HALF)])

    for p in range(TASKS_PER_WORKER):
        t = w * TASKS_PER_WORKER + p
        j = t // 2
        half = t % 2
        _sub_task(uxt_hbm, i1t_hbm, at_hbm, j, half)
        _sub_task(uyt_hbm, i2t_hbm, bt_hbm, j, half)


@jax.jit
def _sc_scatter(uxt, uyt, i1t, i2t):
    return pl.kernel(
        _sc_body,
        mesh=plsc.VectorSubcoreMesh(core_axis_name="c", subcore_axis_name="s"),
        out_type=[
            jax.ShapeDtypeStruct((U, N), jnp.float32),
            jax.ShapeDtypeStruct((U, N), jnp.float32),
        ],
        scratch_types=[
            pltpu.VMEM((HALF,), jnp.float32),
            pltpu.VMEM((ICHUNK,), jnp.int32),
            pltpu.VMEM((ICHUNK,), jnp.int32),
            pltpu.SemaphoreType.DMA((2,)),
        ],
        compiler_params=pltpu.CompilerParams(needs_layout_passes=False),
    )(uxt, uyt, i1t, i2t)


def _tc_post_body(at_ref, bt_ref, out1_ref):
    out1_ref[...] = (at_ref[...] + bt_ref[...]).T


@jax.jit
def _tc_post(at, bt):
    rows = 512
    grid = (N // rows,)
    return pl.pallas_call(
        _tc_post_body,
        grid=grid,
        in_specs=[
            pl.BlockSpec((U, rows), lambda i: (0, i)),
            pl.BlockSpec((U, rows), lambda i: (0, i)),
        ],
        out_specs=pl.BlockSpec((rows, U), lambda i: (i, 0)),
        out_shape=jax.ShapeDtypeStruct((N, U), jnp.float32),
        compiler_params=pltpu.CompilerParams(
            dimension_semantics=("arbitrary",),
        ),
    )(at, bt)


def kernel(unary, deltas, index1, index2):
    uxt, uyt, b, i1t, i2t = _tc_pre(deltas, index1, index2)
    at, bt = _sc_scatter(uxt, uyt, i1t, i2t)
    out1 = _tc_post(at, bt)
    return (out1, b)


# speedup vs baseline: 46.7889x; 1.0008x over previous
import jax
import jax.numpy as jnp
from jax import lax
from jax.experimental import pallas as pl
from jax.experimental.pallas import tpu as pltpu
from jax.experimental.pallas import tpu_sc as plsc

N = 131072
U = 64
NCORES = 2
NSUB = 16
NWORKERS = NCORES * NSUB
HALF = N // 2
TASKS_PER_WORKER = U * 2 // NWORKERS
ICHUNK = 16384
NICHUNK = N // ICHUNK


def _tc_pre_body(deltas_ref, idx1_ref, idx2_ref,
                 uxt_ref, uyt_ref, b_ref, i1t_ref, i2t_ref):
    d = deltas_ref[...]
    uxt_ref[...] = d[:, :U].T
    uyt_ref[...] = d[:, U:2 * U].T
    b_ref[...] = d[:, 2 * U:]
    i1t_ref[...] = idx1_ref[...].T
    i2t_ref[...] = idx2_ref[...].T


@jax.jit
def _tc_pre(deltas, index1, index2):
    rows = 512
    grid = (N // rows,)
    return pl.pallas_call(
        _tc_pre_body,
        grid=grid,
        in_specs=[
            pl.BlockSpec((rows, 3 * U), lambda i: (i, 0)),
            pl.BlockSpec((rows, U), lambda i: (i, 0)),
            pl.BlockSpec((rows, U), lambda i: (i, 0)),
        ],
        out_specs=[
            pl.BlockSpec((U, rows), lambda i: (0, i)),
            pl.BlockSpec((U, rows), lambda i: (0, i)),
            pl.BlockSpec((rows, U), lambda i: (i, 0)),
            pl.BlockSpec((U, rows), lambda i: (0, i)),
            pl.BlockSpec((U, rows), lambda i: (0, i)),
        ],
        out_shape=[
            jax.ShapeDtypeStruct((U, N), jnp.float32),
            jax.ShapeDtypeStruct((U, N), jnp.float32),
            jax.ShapeDtypeStruct((N, U), jnp.float32),
            jax.ShapeDtypeStruct((U, N), jnp.int32),
            jax.ShapeDtypeStruct((U, N), jnp.int32),
        ],
        compiler_params=pltpu.CompilerParams(
            dimension_semantics=("arbitrary",),
        ),
    )(deltas, index1, index2)


def _sc_body(uxt_hbm, uyt_hbm, i1t_hbm, i2t_hbm, at_hbm, bt_hbm,
             dbuf, ibuf0, ibuf1, sems):
    w = lax.axis_index("s") * NCORES + lax.axis_index("c")
    zeros16 = jnp.zeros((16,), jnp.float32)
    ibufs = [ibuf0, ibuf1]

    def _sub_task(src_hbm, idx_hbm, dst_hbm, j, half):
        lo = half * HALF
        pltpu.sync_copy(src_hbm.at[j, pl.ds(lo, HALF)], dbuf)
        h = pltpu.async_copy(idx_hbm.at[j, pl.ds(0, ICHUNK)], ibufs[0],
                             sems.at[0])
        handles = [h, None]
        for k in range(NICHUNK):
            if k + 1 < NICHUNK:
                handles[(k + 1) % 2] = pltpu.async_copy(
                    idx_hbm.at[j, pl.ds((k + 1) * ICHUNK, ICHUNK)],
                    ibufs[(k + 1) % 2], sems.at[(k + 1) % 2])
            handles[k % 2].wait()
            ibuf = ibufs[k % 2]

            def _scan(i, _):
                r = ibuf[pl.ds(i * 16, 16)]
                local = r - lo
                ok = (local >= 0) & (local < HALF)
                plsc.store_scatter(dbuf, [local], zeros16, mask=ok)
                return _
            lax.fori_loop(0, ICHUNK // 16, _scan, None)
        pltpu.sync_copy(dbuf, dst_hbm.at[j, pl.ds(lo, HALF)])

    for p in range(TASKS_PER_WORKER):
        t = w * TASKS_PER_WORKER + p
        j = t // 2
        half = t % 2
        _sub_task(uxt_hbm, i1t_hbm, at_hbm, j, half)
        _sub_task(uyt_hbm, i2t_hbm, bt_hbm, j, half)


@jax.jit
def _sc_scatter(uxt, uyt, i1t, i2t):
    return pl.kernel(
        _sc_body,
        mesh=plsc.VectorSubcoreMesh(core_axis_name="c", subcore_axis_name="s"),
        out_type=[
            jax.ShapeDtypeStruct((U, N), jnp.float32),
            jax.ShapeDtypeStruct((U, N), jnp.float32),
        ],
        scratch_types=[
            pltpu.VMEM((HALF,), jnp.float32),
            pltpu.VMEM((ICHUNK,), jnp.int32),
            pltpu.VMEM((ICHUNK,), jnp.int32),
            pltpu.SemaphoreType.DMA((2,)),
        ],
        compiler_params=pltpu.CompilerParams(
            needs_layout_passes=False, use_tc_tiling_on_sc=True),
    )(uxt, uyt, i1t, i2t)


def _tc_post_body(at_ref, bt_ref, out1_ref):
    out1_ref[...] = (at_ref[...] + bt_ref[...]).T


@jax.jit
def _tc_post(at, bt):
    rows = 512
    grid = (N // rows,)
    return pl.pallas_call(
        _tc_post_body,
        grid=grid,
        in_specs=[
            pl.BlockSpec((U, rows), lambda i: (0, i)),
            pl.BlockSpec((U, rows), lambda i: (0, i)),
        ],
        out_specs=pl.BlockSpec((rows, U), lambda i: (i, 0)),
        out_shape=jax.ShapeDtypeStruct((N, U), jnp.float32),
        compiler_params=pltpu.CompilerParams(
            dimension_semantics=("arbitrary",),
        ),
    )(at, bt)


def kernel(unary, deltas, index1, index2):
    uxt, uyt, b, i1t, i2t = _tc_pre(deltas, index1, index2)
    at, bt = _sc_scatter(uxt, uyt, i1t, i2t)
    out1 = _tc_post(at, bt)
    return (out1, b)


# speedup vs baseline: 72.3137x; 1.5455x over previous
import jax
import jax.numpy as jnp
from jax import lax
from jax.experimental import pallas as pl
from jax.experimental.pallas import tpu as pltpu
from jax.experimental.pallas import tpu_sc as plsc

N = 131072
U = 64
NCORES = 2
NSUB = 16
NWORKERS = NCORES * NSUB
HALF = N // 2
TASKS_PER_WORKER = U * 2 // NWORKERS
ICHUNK = 16384
NICHUNK = N // ICHUNK


def _tc_pre_body(deltas_ref, idx1_ref, idx2_ref,
                 uxt_ref, uyt_ref, b_ref, i1t_ref, i2t_ref):
    d = deltas_ref[...]
    uxt_ref[...] = d[:, :U].T
    uyt_ref[...] = d[:, U:2 * U].T
    b_ref[...] = d[:, 2 * U:]
    i1t_ref[...] = idx1_ref[...].T
    i2t_ref[...] = idx2_ref[...].T


@jax.jit
def _tc_pre(deltas, index1, index2):
    rows = 512
    grid = (N // rows,)
    return pl.pallas_call(
        _tc_pre_body,
        grid=grid,
        in_specs=[
            pl.BlockSpec((rows, 3 * U), lambda i: (i, 0)),
            pl.BlockSpec((rows, U), lambda i: (i, 0)),
            pl.BlockSpec((rows, U), lambda i: (i, 0)),
        ],
        out_specs=[
            pl.BlockSpec((U, rows), lambda i: (0, i)),
            pl.BlockSpec((U, rows), lambda i: (0, i)),
            pl.BlockSpec((rows, U), lambda i: (i, 0)),
            pl.BlockSpec((U, rows), lambda i: (0, i)),
            pl.BlockSpec((U, rows), lambda i: (0, i)),
        ],
        out_shape=[
            jax.ShapeDtypeStruct((U, N), jnp.float32),
            jax.ShapeDtypeStruct((U, N), jnp.float32),
            jax.ShapeDtypeStruct((N, U), jnp.float32),
            jax.ShapeDtypeStruct((U, N), jnp.int32),
            jax.ShapeDtypeStruct((U, N), jnp.int32),
        ],
        compiler_params=pltpu.CompilerParams(
            dimension_semantics=("arbitrary",),
        ),
    )(deltas, index1, index2)


def _sc_body(uxt_hbm, uyt_hbm, i1t_hbm, i2t_hbm, at_hbm, bt_hbm,
             dbuf, ibuf0, ibuf1, sems):
    w = lax.axis_index("s") * NCORES + lax.axis_index("c")
    zeros16 = jnp.zeros((16,), jnp.float32)
    ibufs = [ibuf0, ibuf1]

    def _sub_task(src_hbm, idx_hbm, dst_hbm, j, half):
        lo = half * HALF
        pltpu.sync_copy(src_hbm.at[j, pl.ds(lo, HALF)], dbuf)
        h = pltpu.async_copy(idx_hbm.at[j, pl.ds(0, ICHUNK)], ibufs[0],
                             sems.at[0])
        handles = [h, None]
        for k in range(NICHUNK):
            if k + 1 < NICHUNK:
                handles[(k + 1) % 2] = pltpu.async_copy(
                    idx_hbm.at[j, pl.ds((k + 1) * ICHUNK, ICHUNK)],
                    ibufs[(k + 1) % 2], sems.at[(k + 1) % 2])
            handles[k % 2].wait()
            ibuf = ibufs[k % 2]

            @plsc.parallel_loop(0, ICHUNK // 16, unroll=8)
            def _scan(i):
                r = ibuf[pl.ds(i * 16, 16)]
                local = r - lo
                ok = (local >= 0) & (local < HALF)
                plsc.store_scatter(dbuf, [local], zeros16, mask=ok)
        pltpu.sync_copy(dbuf, dst_hbm.at[j, pl.ds(lo, HALF)])

    for p in range(TASKS_PER_WORKER):
        t = w * TASKS_PER_WORKER + p
        j = t // 2
        half = t % 2
        _sub_task(uxt_hbm, i1t_hbm, at_hbm, j, half)
        _sub_task(uyt_hbm, i2t_hbm, bt_hbm, j, half)


@jax.jit
def _sc_scatter(uxt, uyt, i1t, i2t):
    return pl.kernel(
        _sc_body,
        mesh=plsc.VectorSubcoreMesh(core_axis_name="c", subcore_axis_name="s"),
        out_type=[
            jax.ShapeDtypeStruct((U, N), jnp.float32),
            jax.ShapeDtypeStruct((U, N), jnp.float32),
        ],
        scratch_types=[
            pltpu.VMEM((HALF,), jnp.float32),
            pltpu.VMEM((ICHUNK,), jnp.int32),
            pltpu.VMEM((ICHUNK,), jnp.int32),
            pltpu.SemaphoreType.DMA((2,)),
        ],
        compiler_params=pltpu.CompilerParams(
            needs_layout_passes=False, use_tc_tiling_on_sc=True),
    )(uxt, uyt, i1t, i2t)


def _tc_post_body(at_ref, bt_ref, out1_ref):
    out1_ref[...] = (at_ref[...] + bt_ref[...]).T


@jax.jit
def _tc_post(at, bt):
    rows = 512
    grid = (N // rows,)
    return pl.pallas_call(
        _tc_post_body,
        grid=grid,
        in_specs=[
            pl.BlockSpec((U, rows), lambda i: (0, i)),
            pl.BlockSpec((U, rows), lambda i: (0, i)),
        ],
        out_specs=pl.BlockSpec((rows, U), lambda i: (i, 0)),
        out_shape=jax.ShapeDtypeStruct((N, U), jnp.float32),
        compiler_params=pltpu.CompilerParams(
            dimension_semantics=("arbitrary",),
        ),
    )(at, bt)


def kernel(unary, deltas, index1, index2):
    uxt, uyt, b, i1t, i2t = _tc_pre(deltas, index1, index2)
    at, bt = _sc_scatter(uxt, uyt, i1t, i2t)
    out1 = _tc_post(at, bt)
    return (out1, b)
